# baseline (speedup 1.0000x reference)
import functools

import jax
import jax.numpy as jnp
from jax import lax
from jax.experimental import pallas as pl
from jax.experimental.pallas import tpu as pltpu
from jax.experimental.pallas import tpu_sc as plsc

B, S, D = 4, 8192, 4096
NW = 32
POS_PER_W = S // NW
CH = 8
CHW = CH * D
NCHUNK = POS_PER_W // CH
VECS = CHW // 16
UNROLL = 8


def _sc_body(seq_hbm, tab_hbm, out_hbm, tbuf, sbuf):
    wid = lax.axis_index("s") * 2 + lax.axis_index("c")
    base = wid * POS_PER_W * D

    def chunk(c, _):
        off = base + c * CHW
        pltpu.sync_copy(tab_hbm.at[pl.ds(off, CHW)], tbuf)
        for b in range(B):
            pltpu.sync_copy(seq_hbm.at[b, pl.ds(off, CHW)], sbuf)

            def add(i, _):
                for k in range(UNROLL):
                    j = i * (16 * UNROLL) + k * 16
                    sbuf[pl.ds(j, 16)] = sbuf[pl.ds(j, 16)] + tbuf[pl.ds(j, 16)]
                return 0

            lax.fori_loop(0, VECS // UNROLL, add, 0)
            pltpu.sync_copy(sbuf, out_hbm.at[b, pl.ds(off, CHW)])
        return 0

    lax.fori_loop(0, NCHUNK, chunk, 0)


@jax.jit
def _pos_add_sc(seq2d, tab1d):
    mesh = plsc.VectorSubcoreMesh(core_axis_name="c", subcore_axis_name="s")
    k = functools.partial(
        pl.kernel,
        mesh=mesh,
        out_type=jax.ShapeDtypeStruct((B, S * D), jnp.float32),
        scratch_types=[
            pltpu.VMEM((CHW,), jnp.float32),
            pltpu.VMEM((CHW,), jnp.float32),
        ],
    )(_sc_body)
    return k(seq2d, tab1d)


def kernel(seq, pos_table):
    s = seq.shape[1]
    out2d = _pos_add_sc(
        seq.reshape(B, S * D), pos_table[:s, :].reshape(S * D)
    )
    return out2d.reshape(B, S, D)


# speedup vs baseline: 2.0370x; 2.0370x over previous
import functools

import jax
import jax.numpy as jnp
from jax import lax
from jax.experimental import pallas as pl
from jax.experimental.pallas import tpu as pltpu
from jax.experimental.pallas import tpu_sc as plsc

B, S, D = 4, 8192, 4096
NW = 32
POS_PER_W = S // NW
CH = 8
NCHUNK = POS_PER_W // CH
VECS = CH * D // 16
UNROLL = 8


def _sc_body(seq_hbm, tab_hbm, out_hbm, tbuf, sbuf):
    wid = lax.axis_index("s") * 2 + lax.axis_index("c")
    base = wid * POS_PER_W

    def chunk(c, _):
        s0 = base + c * CH
        pltpu.sync_copy(tab_hbm.at[pl.ds(s0, CH), :], tbuf)
        for b in range(B):
            pltpu.sync_copy(seq_hbm.at[b, pl.ds(s0, CH), :], sbuf)

            def add(i, _):
                for k in range(UNROLL):
                    r = i * UNROLL + k
                    row = r // (D // 16)
                    col = (r % (D // 16)) * 16
                    sbuf[row, pl.ds(col, 16)] = (
                        sbuf[row, pl.ds(col, 16)] + tbuf[row, pl.ds(col, 16)]
                    )
                return 0

            lax.fori_loop(0, VECS // UNROLL, add, 0)
            pltpu.sync_copy(sbuf, out_hbm.at[b, pl.ds(s0, CH), :])
        return 0

    lax.fori_loop(0, NCHUNK, chunk, 0)


@jax.jit
def _pos_add_sc(seq, tab):
    mesh = plsc.VectorSubcoreMesh(core_axis_name="c", subcore_axis_name="s")
    k = functools.partial(
        pl.kernel,
        mesh=mesh,
        out_type=jax.ShapeDtypeStruct((B, S, D), jnp.float32),
        scratch_types=[
            pltpu.VMEM((CH, D), jnp.float32),
            pltpu.VMEM((CH, D), jnp.float32),
        ],
        compiler_params=pltpu.CompilerParams(use_tc_tiling_on_sc=True),
    )(_sc_body)
    return k(seq, tab)


def kernel(seq, pos_table):
    s = seq.shape[1]
    return _pos_add_sc(seq, pos_table[:s, :])


# speedup vs baseline: 4.5137x; 2.2158x over previous
import functools

import jax
import jax.numpy as jnp
from jax import lax
from jax.experimental import pallas as pl
from jax.experimental.pallas import tpu as pltpu
from jax.experimental.pallas import tpu_sc as plsc

B, S, D = 4, 8192, 4096
NW = 32
POS_PER_W = S // NW
CH = 8
NCHUNK = POS_PER_W // CH


def _add_body(seq_ref, tab_ref, out_ref):
    out_ref[...] = seq_ref[...] + tab_ref[...][None, :, :]


def _tc_add(seq, pos_table):
    CHUNK = 128
    grid = (S // CHUNK,)
    return pl.pallas_call(
        _add_body,
        grid=grid,
        in_specs=[
            pl.BlockSpec((B, CHUNK, D), lambda i: (0, i, 0)),
            pl.BlockSpec((CHUNK, D), lambda i: (i, 0)),
        ],
        out_specs=pl.BlockSpec((B, CHUNK, D), lambda i: (0, i, 0)),
        out_shape=jax.ShapeDtypeStruct((B, S, D), seq.dtype),
        compiler_params=pltpu.CompilerParams(
            dimension_semantics=("parallel",),
        ),
    )(seq, pos_table)


def _sc_stream_body(seq_hbm, out_hbm, buf):
    wid = lax.axis_index("s") * 2 + lax.axis_index("c")
    base = wid * POS_PER_W

    def chunk(c, _):
        s0 = base + c * CH
        for b in range(B):
            pltpu.sync_copy(seq_hbm.at[b, pl.ds(s0, CH), :], buf)
        return 0

    lax.fori_loop(0, NCHUNK, chunk, 0)
    pltpu.sync_copy(buf.at[0, pl.ds(0, 16)], out_hbm)


@jax.jit
def _probe(seq, tab):
    mesh = plsc.VectorSubcoreMesh(core_axis_name="c", subcore_axis_name="s")
    sc_dummy = functools.partial(
        pl.kernel,
        mesh=mesh,
        out_type=jax.ShapeDtypeStruct((16,), jnp.float32),
        scratch_types=[pltpu.VMEM((CH, D), jnp.float32)],
        compiler_params=pltpu.CompilerParams(use_tc_tiling_on_sc=True),
    )(_sc_stream_body)(seq)
    out = _tc_add(seq, tab)
    out, _ = lax.optimization_barrier((out, sc_dummy))
    return out


def kernel(seq, pos_table):
    s = seq.shape[1]
    return _probe(seq, pos_table[:s, :])
